# baseline (speedup 1.0000x reference)
import jax
import jax.numpy as jnp
from jax import lax
from jax.experimental import pallas as pl
from jax.experimental.pallas import tpu as pltpu
from jax.experimental.pallas import tpu_sc as plsc

B, S, D = 4, 4096, 1024
NC, NS = 2, 16
NW = NC * NS
SPW = S // NW
RPC = 16
NCH = SPW // RPC
CHUNK = RPC * D
NT = NCH * B
NE = 4
NP = 2


def _sc_body(emb_hbm, pos_hbm, out_hbm, refs):
    (e_bufs, p_bufs, i_sems, o_sems, p_sems) = refs
    wid = lax.axis_index("s") * NC + lax.axis_index("c")
    s_base = wid * SPW

    in_dma = [None] * (NT + 1)
    out_dma = [None] * NT
    pos_dma = [None] * NCH

    pos_dma[0] = pltpu.async_copy(
        pos_hbm.at[pl.ds(s_base, RPC), :], p_bufs[0], p_sems[0])
    in_dma[0] = pltpu.async_copy(
        emb_hbm.at[0, pl.ds(s_base, RPC), :], e_bufs[0], i_sems[0])

    for t in range(NT):
        c, b = divmod(t, B)
        eb = t % NE
        if b == 0:
            pos_dma[c].wait()
            if c + 1 < NCH:
                pos_dma[c + 1] = pltpu.async_copy(
                    pos_hbm.at[pl.ds(s_base + (c + 1) * RPC, RPC), :],
                    p_bufs[(c + 1) % NP], p_sems[(c + 1) % NP])
        in_dma[t].wait()
        if t + 1 < NT:
            if t >= NE - 1:
                out_dma[t - (NE - 1)].wait()
            c1, b1 = divmod(t + 1, B)
            in_dma[t + 1] = pltpu.async_copy(
                emb_hbm.at[b1, pl.ds(s_base + c1 * RPC, RPC), :],
                e_bufs[(t + 1) % NE], i_sems[(t + 1) % NE])

        e = e_bufs[eb]
        p = p_bufs[c % NP]

        @plsc.parallel_loop(0, CHUNK, step=64, unroll=4)
        def add(j, _e=e, _p=p):
            r = lax.shift_right_logical(j, 10)
            col = pl.multiple_of(lax.bitwise_and(j, D - 1), 64)
            for k in range(4):
                ck = pl.multiple_of(col + k * 16, 16)
                plsc.addupdate(_e.at[r, pl.ds(ck, 16)], _p[r, pl.ds(ck, 16)])

        out_dma[t] = pltpu.async_copy(
            e, out_hbm.at[b, pl.ds(s_base + c * RPC, RPC), :], o_sems[eb])

    for t in range(NT - NE, NT):
        out_dma[t].wait()


def kernel(embeddings, pos_table):
    b, s, d = embeddings.shape
    mesh = plsc.VectorSubcoreMesh(core_axis_name="c", subcore_axis_name="s")
    return pl.kernel(
        _sc_body,
        out_type=jax.ShapeDtypeStruct((b, s, d), embeddings.dtype),
        mesh=mesh,
        scratch_types=[(
            tuple(pltpu.VMEM((RPC, D), jnp.float32) for _ in range(NE)),
            tuple(pltpu.VMEM((RPC, D), jnp.float32) for _ in range(NP)),
            tuple(pltpu.SemaphoreType.DMA for _ in range(NE)),
            tuple(pltpu.SemaphoreType.DMA for _ in range(NE)),
            tuple(pltpu.SemaphoreType.DMA for _ in range(NP)),
        )],
    )(embeddings, pos_table[:s])


# speedup vs baseline: 1.0244x; 1.0244x over previous
import jax
import jax.numpy as jnp
from jax import lax
from jax.experimental import pallas as pl
from jax.experimental.pallas import tpu as pltpu
from jax.experimental.pallas import tpu_sc as plsc

B, S, D = 4, 4096, 1024
NC, NS = 2, 16
NW = NC * NS
SPW = S // NW
RPC = 16
NCH = SPW // RPC
CHUNK = RPC * D
NT = NCH * B
NE = 4
NP = 2


def _sc_body(emb_hbm, pos_hbm, out_hbm, refs):
    (e_bufs, p_bufs, i_sems, o_sems, p_sems) = refs
    wid = lax.axis_index("s") * NC + lax.axis_index("c")
    s_base = wid * SPW

    in_dma = [None] * (NT + 1)
    out_dma = [None] * NT
    pos_dma = [None] * NCH

    pos_dma[0] = pltpu.async_copy(
        pos_hbm.at[pl.ds(s_base, RPC), :], p_bufs[0], p_sems[0])
    in_dma[0] = pltpu.async_copy(
        emb_hbm.at[0, pl.ds(s_base, RPC), :], e_bufs[0], i_sems[0])

    for t in range(NT):
        c, b = divmod(t, B)
        eb = t % NE
        if b == 0:
            pos_dma[c].wait()
            if c + 1 < NCH:
                pos_dma[c + 1] = pltpu.async_copy(
                    pos_hbm.at[pl.ds(s_base + (c + 1) * RPC, RPC), :],
                    p_bufs[(c + 1) % NP], p_sems[(c + 1) % NP])
        in_dma[t].wait()
        if t + 1 < NT:
            if t >= NE - 1:
                out_dma[t - (NE - 1)].wait()
            c1, b1 = divmod(t + 1, B)
            in_dma[t + 1] = pltpu.async_copy(
                emb_hbm.at[b1, pl.ds(s_base + c1 * RPC, RPC), :],
                e_bufs[(t + 1) % NE], i_sems[(t + 1) % NE])

        e = e_bufs[eb]
        p = p_bufs[c % NP]

        @plsc.parallel_loop(0, CHUNK, step=64, unroll=2)
        def add(j, _e=e, _p=p):
            r = lax.shift_right_logical(j, 10)
            col = pl.multiple_of(lax.bitwise_and(j, D - 1), 64)
            for k in range(4):
                ck = pl.multiple_of(col + k * 16, 16)
                plsc.addupdate(_e.at[r, pl.ds(ck, 16)], _p[r, pl.ds(ck, 16)])

        out_dma[t] = pltpu.async_copy(
            e, out_hbm.at[b, pl.ds(s_base + c * RPC, RPC), :], o_sems[eb])

    for t in range(NT - NE, NT):
        out_dma[t].wait()


def kernel(embeddings, pos_table):
    b, s, d = embeddings.shape
    mesh = plsc.VectorSubcoreMesh(core_axis_name="c", subcore_axis_name="s")
    return pl.kernel(
        _sc_body,
        out_type=jax.ShapeDtypeStruct((b, s, d), embeddings.dtype),
        mesh=mesh,
        scratch_types=[(
            tuple(pltpu.VMEM((RPC, D), jnp.float32) for _ in range(NE)),
            tuple(pltpu.VMEM((RPC, D), jnp.float32) for _ in range(NP)),
            tuple(pltpu.SemaphoreType.DMA for _ in range(NE)),
            tuple(pltpu.SemaphoreType.DMA for _ in range(NE)),
            tuple(pltpu.SemaphoreType.DMA for _ in range(NP)),
        )],
    )(embeddings, pos_table[:s])
